# baseline (speedup 1.0000x reference)
import functools

import jax
import jax.numpy as jnp
from jax import lax
from jax.experimental import pallas as pl
from jax.experimental.pallas import tpu as pltpu
from jax.experimental.pallas import tpu_sc as plsc

_N_CAT = 26
_VOCAB = 100000
_EMBED = 32
_BATCH = 16384
_DENSE_DIM = 13
_NC = 2
_NS = 16
_NW = _NC * _NS
_BPW = _BATCH // _NW
_CH = 128
_NCHUNK = _BPW // _CH
_SPLIT = 13


def _make_half(nfeat, with_dense):
    nslots = nfeat + (1 if with_dense else 0)

    def body(idx_hbm, tab_hbm, dense_t_hbm, w_hbm, b_hbm, out_hbm,
             idx_v, slab_v, dt_v, w_v, b_v, gsem, ssem):
        c = lax.axis_index("c")
        s = lax.axis_index("s")
        wid = s * _NC + c
        base = wid * _BPW

        def fetch_idx(f, p):
            pltpu.sync_copy(idx_hbm.at[f, pl.ds(base, _BPW)], idx_v.at[p])

        def fire_feature(f, p):
            def per_dim(e, carry):
                row = f * _EMBED + e
                for ci in range(_NCHUNK):
                    pltpu.async_copy(
                        tab_hbm.at[row].at[idx_v.at[p, pl.ds(ci * _CH, _CH)]],
                        slab_v.at[p, e // 8, ci, lax.rem(e, 8), :],
                        gsem.at[p],
                    )
                return carry

            lax.fori_loop(0, _EMBED, per_dim, 0)

        def drain(sem_arr, p):
            pltpu.make_async_copy(
                out_hbm.at[pl.ds(0, 1), :, pl.ds(0, _NCHUNK)],
                slab_v.at[pl.ds(p, 1)],
                sem_arr.at[p],
            ).wait()

        def scatter(f, p):
            pltpu.async_copy(
                slab_v.at[pl.ds(p, 1)],
                out_hbm.at[pl.ds(f, 1), :, pl.ds(wid * _NCHUNK, _NCHUNK)],
                ssem.at[p],
            )

        fetch_idx(0, 0)
        fire_feature(0, 0)

        def step(f, carry):
            p0 = lax.rem(f, 2)
            p1 = lax.rem(f + 1, 2)

            @pl.when(f + 1 < nfeat)
            def _():
                fetch_idx(f + 1, p1)

                @pl.when(f >= 1)
                def _():
                    drain(ssem, p1)

                fire_feature(f + 1, p1)

            drain(gsem, p0)
            scatter(f, p0)
            return carry

        lax.fori_loop(0, nfeat, step, 0)

        if with_dense:
            pltpu.sync_copy(dense_t_hbm.at[:, pl.ds(base, _BPW)], dt_v)
            pltpu.sync_copy(w_hbm, w_v)
            pltpu.sync_copy(b_hbm, b_v)
            drain(ssem, lax.rem(nfeat, 2))
            w_vecs = [w_v[pl.ds(i * 16, 16)]
                      for i in range(_DENSE_DIM * _EMBED // 16)]
            b_vecs = [b_v[pl.ds(i * 16, 16)] for i in range(_EMBED // 16)]
            pd = lax.rem(nfeat, 2)

            def per_group(g, carry):
                col = g * 16
                d = [dt_v[k, pl.ds(col, 16)] for k in range(_DENSE_DIM)]
                for e in range(_EMBED):
                    acc = b_vecs[e // 16][e % 16] + jnp.zeros((16,), jnp.float32)
                    for k in range(_DENSE_DIM):
                        i = k * _EMBED + e
                        acc = acc + d[k] * w_vecs[i // 16][i % 16]
                    slab_v[pd, e // 8, g // 8, lax.rem(e, 8),
                           pl.ds(lax.rem(g, 8) * 16, 16)] = jnp.maximum(acc, 0.0)
                return carry

            lax.fori_loop(0, _BPW // 16, per_group, 0)
            scatter(nfeat, pd)
            drain(ssem, lax.rem(nfeat + 1, 2))
            drain(ssem, pd)
        else:
            drain(ssem, lax.rem(nfeat + 1, 2))
            drain(ssem, lax.rem(nfeat, 2))

        return

    return pl.kernel(
        body,
        mesh=plsc.VectorSubcoreMesh(core_axis_name="c", subcore_axis_name="s"),
        out_type=jax.ShapeDtypeStruct((nslots, 4, _BATCH // 128, 8, 128),
                                      jnp.float32),
        scratch_types=[
            pltpu.VMEM((2, _BPW), jnp.int32),
            pltpu.VMEM((2, 4, _NCHUNK, 8, 128), jnp.float32),
            pltpu.VMEM((_DENSE_DIM, _BPW), jnp.float32),
            pltpu.VMEM((_DENSE_DIM * _EMBED,), jnp.float32),
            pltpu.VMEM((_EMBED,), jnp.float32),
            pltpu.SemaphoreType.DMA((2,)),
            pltpu.SemaphoreType.DMA((2,)),
        ],
        compiler_params=pltpu.CompilerParams(use_tc_tiling_on_sc=False),
    )


_half_a = _make_half(_SPLIT, False)
_half_b = _make_half(_N_CAT - _SPLIT, True)


def kernel(cat_0, cat_1, cat_2, cat_3, cat_4, cat_5, cat_6, cat_7, cat_8,
           cat_9, cat_10, cat_11, cat_12, cat_13, cat_14, cat_15, cat_16,
           cat_17, cat_18, cat_19, cat_20, cat_21, cat_22, cat_23, cat_24,
           cat_25, dense_0, tables, W, b):
    cats = [cat_0, cat_1, cat_2, cat_3, cat_4, cat_5, cat_6, cat_7, cat_8,
            cat_9, cat_10, cat_11, cat_12, cat_13, cat_14, cat_15, cat_16,
            cat_17, cat_18, cat_19, cat_20, cat_21, cat_22, cat_23, cat_24,
            cat_25]
    idx_a = jnp.stack(cats[:_SPLIT], axis=0)
    idx_b = jnp.stack(cats[_SPLIT:], axis=0)
    tab_t = tables.transpose(0, 2, 1)
    tab_a = tab_t[:_SPLIT].reshape(_SPLIT * _EMBED, _VOCAB)
    tab_b = tab_t[_SPLIT:].reshape((_N_CAT - _SPLIT) * _EMBED, _VOCAB)
    dense_t = dense_0.T
    w1 = W.reshape(-1)
    o_a = _half_a(idx_a, tab_a, dense_t, w1, b)
    o_b = _half_b(idx_b, tab_b, dense_t, w1, b)
    out5 = jnp.concatenate([o_a, o_b], axis=0)
    return out5.transpose(2, 4, 0, 1, 3).reshape(_BATCH, _N_CAT + 1, _EMBED)
